# baseline (speedup 1.0000x reference)
import functools

import jax
import jax.numpy as jnp
import numpy as np
from jax import lax
from jax.experimental import pallas as pl
from jax.experimental.pallas import tpu as pltpu
from jax.experimental.pallas import tpu_sc as plsc

N = 10000
E = 320000
D = 128
ED = 16
L = 4
H = 8
V = 256
U = 150
DH = D // H
CE = 128
NB = 16
NT = 32
NBL = N // NB
ROWP = N + 16

_pos = np.arange(D)
PERM_T = ((_pos % H) * DH + _pos // H).astype(np.int32)

_lane = np.arange(16)
IDX_TAB = np.zeros((16, 16), np.int32)
IDX_TAB[0] = 8 + (_lane % 8)
IDX_TAB[1] = _lane % 8
for _ii in range(8):
    IDX_TAB[2 + _ii] = 2 * _ii + _lane // 8


def _we2(We):
    c = np.arange(D)
    d = np.arange(D)
    mask = (d[:, None] // DH) == (c[None, :] // DH)
    A = We[c % ED, :]
    return A.T * mask


def _we3(We):
    r = np.arange(D)
    c = np.arange(D)
    mask = (r[:, None] // DH) == (c[None, :] // DH)
    A2 = We[r % ED, :]
    return A2 * mask


def _fold_weights(Wq, Wk, Wv, We, Wo):
    We2 = _we2(We)
    WqT = (0.25 * Wq)[:, PERM_T]
    WPT = (0.25 * (Wq @ We2))[:, PERM_T]
    WQP = jnp.concatenate([WqT, WPT], axis=1)
    WKV = jnp.concatenate([Wk[:, PERM_T], Wv[:, PERM_T]], axis=1)
    WB = jnp.concatenate([Wo[PERM_T, :], (_we3(We) @ Wo)[PERM_T, :]], axis=0)
    return WQP, WKV, WB


def _tc_proj_body(x_ref, wqp_ref, wkv_ref, qp_ref, kv_ref):
    x = x_ref[...]
    qp_ref[...] = jnp.dot(x, wqp_ref[...], preferred_element_type=jnp.float32)
    kv_ref[...] = jnp.dot(x, wkv_ref[...], preferred_element_type=jnp.float32)


def _tc_proj(x, WQP, WKV):
    BN_ = 1000
    return pl.pallas_call(
        _tc_proj_body,
        grid=(N // BN_,),
        in_specs=[pl.BlockSpec((BN_, D), lambda i: (i, 0)),
                  pl.BlockSpec((D, 2 * D), lambda i: (0, 0)),
                  pl.BlockSpec((D, 2 * D), lambda i: (0, 0))],
        out_specs=[pl.BlockSpec((BN_, 2 * D), lambda i: (i, 0)),
                   pl.BlockSpec((BN_, 2 * D), lambda i: (i, 0))],
        out_shape=[jax.ShapeDtypeStruct((N, 2 * D), jnp.float32),
                   jax.ShapeDtypeStruct((N, 2 * D), jnp.float32)],
    )(x, WQP, WKV)


def _tc_emb_body(ut_ref, tab_ref, tqp_ref, tkv_ref, x0_ref, qp_ref, kv_ref):
    oh = (ut_ref[...] == lax.broadcasted_iota(jnp.int32, (ut_ref.shape[0], V), 1)
          ).astype(jnp.float32)
    x0_ref[...] = jnp.dot(oh, tab_ref[...], preferred_element_type=jnp.float32)
    qp_ref[...] = jnp.dot(oh, tqp_ref[...], preferred_element_type=jnp.float32)
    kv_ref[...] = jnp.dot(oh, tkv_ref[...], preferred_element_type=jnp.float32)


def _tc_emb(ut2, TAB, TQP, TKV):
    BN_ = 1000
    return pl.pallas_call(
        _tc_emb_body,
        grid=(N // BN_,),
        in_specs=[pl.BlockSpec((BN_, 1), lambda i: (i, 0)),
                  pl.BlockSpec((V, D), lambda i: (0, 0)),
                  pl.BlockSpec((V, 2 * D), lambda i: (0, 0)),
                  pl.BlockSpec((V, 2 * D), lambda i: (0, 0))],
        out_specs=[pl.BlockSpec((BN_, D), lambda i: (i, 0)),
                   pl.BlockSpec((BN_, 2 * D), lambda i: (i, 0)),
                   pl.BlockSpec((BN_, 2 * D), lambda i: (i, 0))],
        out_shape=[jax.ShapeDtypeStruct((N, D), jnp.float32),
                   jax.ShapeDtypeStruct((N, 2 * D), jnp.float32),
                   jax.ShapeDtypeStruct((N, 2 * D), jnp.float32)],
    )(ut2, TAB, TQP, TKV)


def _tc_out_body(aggb_ref, x_ref, wb_ref, xn_ref):
    y = jnp.dot(aggb_ref[...], wb_ref[...], preferred_element_type=jnp.float32)
    xn_ref[...] = jnp.maximum(y, 0.0) + x_ref[...]


def _tc_out(aggb, x, WB):
    BN_ = 1000
    return pl.pallas_call(
        _tc_out_body,
        grid=(N // BN_,),
        in_specs=[pl.BlockSpec((BN_, 2 * D), lambda i: (i, 0)),
                  pl.BlockSpec((BN_, D), lambda i: (i, 0)),
                  pl.BlockSpec((2 * D, D), lambda i: (0, 0))],
        out_specs=pl.BlockSpec((BN_, D), lambda i: (i, 0)),
        out_shape=jax.ShapeDtypeStruct((N, D), jnp.float32),
    )(aggb, x, WB)


def _readout_body(xa_ref, xm_ref, gf_ref, nf_ref):
    i = pl.program_id(0)

    @pl.when(i == 0)
    def _init():
        gf_ref[...] = jnp.zeros_like(gf_ref)

    a = xa_ref[...]
    m = xm_ref[...]
    nf_ref[:, 0:D] = a
    nf_ref[:, D:2 * D] = m
    gf_ref[:, 0:D] += a.sum(axis=0, keepdims=True)
    gf_ref[:, D:2 * D] += m.sum(axis=0, keepdims=True)


def _readout(xa, xm):
    BN_ = 1000
    return pl.pallas_call(
        _readout_body,
        grid=(N // BN_,),
        in_specs=[pl.BlockSpec((BN_, D), lambda i: (i, 0)),
                  pl.BlockSpec((BN_, D), lambda i: (i, 0))],
        out_specs=[pl.BlockSpec((1, 2 * D), lambda i: (0, 0)),
                   pl.BlockSpec((BN_, 2 * D), lambda i: (i, 0))],
        out_shape=[jax.ShapeDtypeStruct((1, 2 * D), jnp.float32),
                   jax.ShapeDtypeStruct((N, 2 * D), jnp.float32)],
    )(xa, xm)


def _vgather(v, idx):
    return lax.gather(
        v, idx[:, None],
        lax.GatherDimensionNumbers(offset_dims=(), collapsed_slice_dims=(0,),
                                   start_index_map=(0,)),
        (1,), mode=lax.GatherScatterMode.PROMISE_IN_BOUNDS)


CMAX = E // CE - 1


def _sc_edge_body(qp_hbm, kv_hbm, ef_hbm, src_hbm, rowlo_hbm, rowhi_hbm, bounds_hbm,
             idx_hbm, out_hbm, rowlo_v, rowhi_v, bounds_v, idx_v, src_v, kv_v,
             ef_v, qp_v, out_v, sem0, sem1):
    wid = lax.axis_index("s") * 2 + lax.axis_index("c")
    pltpu.sync_copy(rowlo_hbm, rowlo_v)
    pltpu.sync_copy(rowhi_hbm, rowhi_v)
    pltpu.sync_copy(bounds_hbm, bounds_v)
    pltpu.sync_copy(idx_hbm, idx_v)
    bv = bounds_v[pl.ds(wid, 16)]
    b0 = bv[0]
    b1 = bv[1]

    idx_hi = idx_v[0, :]
    idx_lo2 = idx_v[1, :]
    idxp = [idx_v[2 + ii, :] for ii in range(8)]

    def _drain(pref, buf, sem):
        pltpu.make_async_copy(kv_hbm.at[src_v.at[buf]], kv_v.at[buf], sem).wait()
        pltpu.make_async_copy(ef_hbm.at[pl.ds(pref * CE, CE)], ef_v.at[buf],
                              sem).wait()

    def _cold_load(c, buf):
        pltpu.sync_copy(src_hbm.at[pl.ds(c * CE, CE)], src_v.at[buf])
        pltpu.sync_copy(kv_hbm.at[src_v.at[buf]], kv_v.at[buf])
        pltpu.sync_copy(ef_hbm.at[pl.ds(c * CE, CE)], ef_v.at[buf])

    def _prefetch(c1, buf, sem):
        pltpu.sync_copy(src_hbm.at[pl.ds(c1 * CE, CE)], src_v.at[buf])
        pltpu.async_copy(kv_hbm.at[src_v.at[buf]], kv_v.at[buf], sem)
        pltpu.async_copy(ef_hbm.at[pl.ds(c1 * CE, CE)], ef_v.at[buf], sem)

    def b_body(b, bc):
        pltpu.sync_copy(qp_hbm.at[pl.ds(b * NB, NB)], qp_v)

        def r_body(r, bc):
            n = b * NB + r
            jlo = rowlo_v[pl.ds(n, 16)][0]
            jhi = rowhi_v[pl.ds(n, 16)][0]
            qregs = [qp_v[r, pl.ds(ii * 16, 16)] for ii in range(16)]

            zero = jnp.zeros((16,), jnp.float32)
            minit = jnp.full((16,), -1e30, jnp.float32)
            init = bc + (minit, zero) + tuple([zero] * 16)

            def e_body(j, carry):
                loaded_c, pref, m, den = carry[0], carry[1], carry[2], carry[3]
                accs = carry[4:]
                c = j // CE

                @pl.when(c != loaded_c)
                def _switch():
                    @pl.when(pref >= 0)
                    def _():
                        @pl.when((pref & 1) == 0)
                        def _():
                            _drain(pref, 0, sem0)

                        @pl.when((pref & 1) == 1)
                        def _():
                            _drain(pref, 1, sem1)

                    @pl.when(pref != c)
                    def _():
                        @pl.when((c & 1) == 0)
                        def _():
                            _cold_load(c, 0)

                        @pl.when((c & 1) == 1)
                        def _():
                            _cold_load(c, 1)

                    @pl.when(c + 1 <= CMAX)
                    def _():
                        @pl.when((c & 1) == 0)
                        def _():
                            _prefetch(c + 1, 1, sem1)

                        @pl.when((c & 1) == 1)
                        def _():
                            _prefetch(c + 1, 0, sem0)

                new_pref = jnp.where(c != loaded_c,
                                     jnp.where(c + 1 <= CMAX, c + 1,
                                               jnp.int32(-2)),
                                     pref)

                bb = c & 1
                i = j - c * CE
                efv = ef_v[bb, i, :]
                eps = [_vgather(efv, idxp[ii]) for ii in range(8)]
                sacc = zero
                for ii in range(8):
                    sacc = sacc + kv_v[bb, i, pl.ds(ii * 16, 16)] * qregs[ii]
                for ii in range(8):
                    sacc = sacc + eps[ii] * qregs[8 + ii]
                s = sacc + _vgather(sacc, idx_hi)

                mn = jnp.maximum(m, s)
                scale = jnp.exp(m - mn)
                p = jnp.exp(s - mn)
                den2 = den * scale + p
                pp = _vgather(p, idx_lo2)
                scp = _vgather(scale, idx_lo2)
                naccs = []
                for ii in range(8):
                    naccs.append(accs[ii] * scp
                                 + kv_v[bb, i, pl.ds(D + ii * 16, 16)] * pp)
                for ii in range(8):
                    naccs.append(accs[8 + ii] * scp + eps[ii] * pp)
                return (c, new_pref, mn, den2) + tuple(naccs)

            fin = lax.fori_loop(jlo, jhi, e_body, init)
            den = fin[3]
            accs = fin[4:]
            rden = 1.0 / (den + 1e-9)
            rpp = _vgather(rden, idx_lo2)
            for ii in range(16):
                out_v[r, pl.ds(ii * 16, 16)] = accs[ii] * rpp
            return (fin[0], fin[1])

        bc = lax.fori_loop(0, NB, r_body, bc)
        pltpu.sync_copy(out_v, out_hbm.at[pl.ds(b * NB, NB)])
        return bc

    bcf = lax.fori_loop(b0, b1, b_body, (jnp.int32(-1), jnp.int32(-2)))
    pref_end = bcf[1]

    @pl.when(pref_end >= 0)
    def _final_drain():
        @pl.when((pref_end & 1) == 0)
        def _():
            _drain(pref_end, 0, sem0)

        @pl.when((pref_end & 1) == 1)
        def _():
            _drain(pref_end, 1, sem1)


@functools.lru_cache(maxsize=None)
def _sc_edge():
    mesh = plsc.VectorSubcoreMesh(core_axis_name="c", subcore_axis_name="s")
    return pl.kernel(
        _sc_edge_body,
        mesh=mesh,
        out_type=jax.ShapeDtypeStruct((N, 2 * D), jnp.float32),
        scratch_types=[
            pltpu.VMEM((ROWP,), jnp.int32),
            pltpu.VMEM((ROWP,), jnp.int32),
            pltpu.VMEM((48,), jnp.int32),
            pltpu.VMEM((16, 16), jnp.int32),
            pltpu.VMEM((2, CE), jnp.int32),
            pltpu.VMEM((2, CE, 2 * D), jnp.float32),
            pltpu.VMEM((2, CE, ED), jnp.float32),
            pltpu.VMEM((NB, 2 * D), jnp.float32),
            pltpu.VMEM((NB, 2 * D), jnp.float32),
            pltpu.SemaphoreType.DMA,
            pltpu.SemaphoreType.DMA,
        ],
    )


KCH = 2000


def _sc_key_body(ut_hbm, src_hbm, dst_hbm, key_hbm, ut_v, src_v, dst_v, key_v):
    wid = lax.axis_index("s") * 2 + lax.axis_index("c")
    pltpu.sync_copy(ut_hbm, ut_v)
    base = wid * (E // NT)
    for ch in range(E // NT // KCH):
        off = base + ch * KCH
        pltpu.sync_copy(src_hbm.at[pl.ds(off, KCH)], src_v)
        pltpu.sync_copy(dst_hbm.at[pl.ds(off, KCH)], dst_v)

        def step(v, _):
            sidx = src_v[pl.ds(v * 16, 16)]
            didx = dst_v[pl.ds(v * 16, 16)]
            uts = plsc.load_gather(ut_v, [sidx])
            utd = plsc.load_gather(ut_v, [didx])
            ms = (uts < U).astype(jnp.int32)
            md = (utd < U).astype(jnp.int32)
            key = didx * 2 + (1 - ms * md)
            key_v[pl.ds(v * 16, 16)] = key
            return _

        lax.fori_loop(0, KCH // 16, step, jnp.int32(0))
        pltpu.sync_copy(key_v, key_hbm.at[pl.ds(off, KCH)])


@functools.lru_cache(maxsize=None)
def _sc_key():
    mesh = plsc.VectorSubcoreMesh(core_axis_name="c", subcore_axis_name="s")
    return pl.kernel(
        _sc_key_body,
        mesh=mesh,
        out_type=jax.ShapeDtypeStruct((E,), jnp.int32),
        scratch_types=[
            pltpu.VMEM((N,), jnp.int32),
            pltpu.VMEM((KCH,), jnp.int32),
            pltpu.VMEM((KCH,), jnp.int32),
            pltpu.VMEM((KCH,), jnp.int32),
        ],
    )


def _bounds(cum_end, total):
    targets = (jnp.arange(33, dtype=jnp.int32) * total) // 32
    bnd = jnp.searchsorted(cum_end, targets, side="left").astype(jnp.int32) // NB
    bnd = bnd.at[0].set(0).at[32].set(NBL)
    bnd = jnp.concatenate([bnd, jnp.zeros((15,), jnp.int32)])
    return bnd


def kernel(input, unit_type, edge_index, edge_feature, emb_mono, emb_atom,
           Wq_a, Wk_a, Wv_a, Wo_a, We_a, Wq_m, Wk_m, Wv_m, Wo_m, We_m):
    src = edge_index[0].astype(jnp.int32)
    dst = edge_index[1].astype(jnp.int32)
    ut = unit_type.astype(jnp.int32)

    utm = (ut < U).astype(jnp.int32)
    key2 = dst * 2 + (1 - utm[src] * utm[dst])
    eid = jnp.arange(E, dtype=jnp.int32)
    key2s, srcP, eidP = lax.sort((key2, src, eid), num_keys=1)
    efP = edge_feature[eidP]
    pos = jnp.searchsorted(key2s, jnp.arange(2 * N + 2, dtype=jnp.int32)
                           ).astype(jnp.int32)
    rowlo = pos[0:2 * N:2]
    rowhiM = pos[1:2 * N + 1:2]
    cumA = pos[2:2 * N + 2:2]
    cumM = jnp.cumsum(rowhiM - rowlo).astype(jnp.int32)
    bndA = _bounds(cumA, jnp.int32(E))
    bndM = _bounds(cumM, cumM[N - 1])
    zpad = jnp.zeros((16,), jnp.int32)
    rowlo_p = jnp.concatenate([rowlo, zpad])
    cumA_p = jnp.concatenate([cumA, zpad])
    rowhiM_p = jnp.concatenate([rowhiM, zpad])
    idxtab = jnp.asarray(IDX_TAB)

    WQPs, WKVs, WBs = [], [], []
    for i in range(L):
        for (Wq, Wk, Wv, We, Wo) in ((Wq_a[i], Wk_a[i], Wv_a[i], We_a[i], Wo_a[i]),
                                     (Wq_m[i], Wk_m[i], Wv_m[i], We_m[i], Wo_m[i])):
            WQP, WKV, WB = _fold_weights(Wq, Wk, Wv, We, Wo)
            WQPs.append(WQP)
            WKVs.append(WKV)
            WBs.append(WB)

    ut2 = ut.reshape(N, 1)
    TQP_a = emb_atom @ WQPs[0]
    TKV_a = emb_atom @ WKVs[0]
    TQP_m = emb_mono @ WQPs[1]
    TKV_m = emb_mono @ WKVs[1]
    xa, qpa, kva = _tc_emb(ut2, emb_atom, TQP_a, TKV_a)
    xm, qpm, kvm = _tc_emb(ut2, emb_mono, TQP_m, TKV_m)

    for i in range(L):
        if i > 0:
            qpa, kva = _tc_proj(xa, WQPs[2 * i], WKVs[2 * i])
            qpm, kvm = _tc_proj(xm, WQPs[2 * i + 1], WKVs[2 * i + 1])
        agga = _sc_edge()(qpa, kva, efP, srcP, rowlo_p, cumA_p, bndA, idxtab)
        aggm = _sc_edge()(qpm, kvm, efP, srcP, rowlo_p, rowhiM_p, bndM, idxtab)
        xa = _tc_out(agga, xa, WBs[2 * i])
        xm = _tc_out(aggm, xm, WBs[2 * i + 1])

    gf, nf = _readout(xa, xm)
    return (gf, nf)


# speedup vs baseline: 1.2001x; 1.2001x over previous
import functools

import jax
import jax.numpy as jnp
import numpy as np
from jax import lax
from jax.experimental import pallas as pl
from jax.experimental.pallas import tpu as pltpu
from jax.experimental.pallas import tpu_sc as plsc

N = 10000
E = 320000
D = 128
ED = 16
L = 4
H = 8
V = 256
U = 150
DH = D // H
CE = 128
NB = 16
NT = 32
NBL = N // NB
ROWP = N + 16

_pos = np.arange(D)
PERM_T = ((_pos % H) * DH + _pos // H).astype(np.int32)

_lane = np.arange(16)
IDX_TAB = np.zeros((16, 16), np.int32)
IDX_TAB[0] = 8 + (_lane % 8)
IDX_TAB[1] = _lane % 8
for _ii in range(8):
    IDX_TAB[2 + _ii] = 2 * _ii + _lane // 8


def _we2(We):
    c = np.arange(D)
    d = np.arange(D)
    mask = (d[:, None] // DH) == (c[None, :] // DH)
    A = We[c % ED, :]
    return A.T * mask


def _we3(We):
    r = np.arange(D)
    c = np.arange(D)
    mask = (r[:, None] // DH) == (c[None, :] // DH)
    A2 = We[r % ED, :]
    return A2 * mask


def _fold_weights(Wq, Wk, Wv, We, Wo):
    We2 = _we2(We)
    WqT = (0.25 * Wq)[:, PERM_T]
    WPT = (0.25 * (Wq @ We2))[:, PERM_T]
    WQP = jnp.concatenate([WqT, WPT], axis=1)
    WKV = jnp.concatenate([Wk[:, PERM_T], Wv[:, PERM_T]], axis=1)
    WB = jnp.concatenate([Wo[PERM_T, :], (_we3(We) @ Wo)[PERM_T, :]], axis=0)
    return WQP, WKV, WB


def _tc_proj_body(x_ref, wqp_ref, wkv_ref, qp_ref, kv_ref):
    x = x_ref[...]
    qp_ref[...] = jnp.dot(x, wqp_ref[...], preferred_element_type=jnp.float32)
    kv_ref[...] = jnp.dot(x, wkv_ref[...], preferred_element_type=jnp.float32)


def _tc_proj(x, WQP, WKV):
    BN_ = 1000
    return pl.pallas_call(
        _tc_proj_body,
        grid=(N // BN_,),
        in_specs=[pl.BlockSpec((BN_, D), lambda i: (i, 0)),
                  pl.BlockSpec((D, 2 * D), lambda i: (0, 0)),
                  pl.BlockSpec((D, 2 * D), lambda i: (0, 0))],
        out_specs=[pl.BlockSpec((BN_, 2 * D), lambda i: (i, 0)),
                   pl.BlockSpec((BN_, 2 * D), lambda i: (i, 0))],
        out_shape=[jax.ShapeDtypeStruct((N, 2 * D), jnp.float32),
                   jax.ShapeDtypeStruct((N, 2 * D), jnp.float32)],
    )(x, WQP, WKV)


def _tc_emb_body(ut_ref, tab_ref, tqp_ref, tkv_ref, x0_ref, qp_ref, kv_ref):
    oh = (ut_ref[...] == lax.broadcasted_iota(jnp.int32, (ut_ref.shape[0], V), 1)
          ).astype(jnp.float32)
    x0_ref[...] = jnp.dot(oh, tab_ref[...], preferred_element_type=jnp.float32)
    qp_ref[...] = jnp.dot(oh, tqp_ref[...], preferred_element_type=jnp.float32)
    kv_ref[...] = jnp.dot(oh, tkv_ref[...], preferred_element_type=jnp.float32)


def _tc_emb(ut2, TAB, TQP, TKV):
    BN_ = 1000
    return pl.pallas_call(
        _tc_emb_body,
        grid=(N // BN_,),
        in_specs=[pl.BlockSpec((BN_, 1), lambda i: (i, 0)),
                  pl.BlockSpec((V, D), lambda i: (0, 0)),
                  pl.BlockSpec((V, 2 * D), lambda i: (0, 0)),
                  pl.BlockSpec((V, 2 * D), lambda i: (0, 0))],
        out_specs=[pl.BlockSpec((BN_, D), lambda i: (i, 0)),
                   pl.BlockSpec((BN_, 2 * D), lambda i: (i, 0)),
                   pl.BlockSpec((BN_, 2 * D), lambda i: (i, 0))],
        out_shape=[jax.ShapeDtypeStruct((N, D), jnp.float32),
                   jax.ShapeDtypeStruct((N, 2 * D), jnp.float32),
                   jax.ShapeDtypeStruct((N, 2 * D), jnp.float32)],
    )(ut2, TAB, TQP, TKV)


def _tc_out_body(aggb_ref, x_ref, wb_ref, xn_ref):
    y = jnp.dot(aggb_ref[...], wb_ref[...], preferred_element_type=jnp.float32)
    xn_ref[...] = jnp.maximum(y, 0.0) + x_ref[...]


def _tc_out(aggb, x, WB):
    BN_ = 1000
    return pl.pallas_call(
        _tc_out_body,
        grid=(N // BN_,),
        in_specs=[pl.BlockSpec((BN_, 2 * D), lambda i: (i, 0)),
                  pl.BlockSpec((BN_, D), lambda i: (i, 0)),
                  pl.BlockSpec((2 * D, D), lambda i: (0, 0))],
        out_specs=pl.BlockSpec((BN_, D), lambda i: (i, 0)),
        out_shape=jax.ShapeDtypeStruct((N, D), jnp.float32),
    )(aggb, x, WB)


def _readout_body(xa_ref, xm_ref, gf_ref, nf_ref):
    i = pl.program_id(0)

    @pl.when(i == 0)
    def _init():
        gf_ref[...] = jnp.zeros_like(gf_ref)

    a = xa_ref[...]
    m = xm_ref[...]
    nf_ref[:, 0:D] = a
    nf_ref[:, D:2 * D] = m
    gf_ref[:, 0:D] += a.sum(axis=0, keepdims=True)
    gf_ref[:, D:2 * D] += m.sum(axis=0, keepdims=True)


def _readout(xa, xm):
    BN_ = 1000
    return pl.pallas_call(
        _readout_body,
        grid=(N // BN_,),
        in_specs=[pl.BlockSpec((BN_, D), lambda i: (i, 0)),
                  pl.BlockSpec((BN_, D), lambda i: (i, 0))],
        out_specs=[pl.BlockSpec((1, 2 * D), lambda i: (0, 0)),
                   pl.BlockSpec((BN_, 2 * D), lambda i: (i, 0))],
        out_shape=[jax.ShapeDtypeStruct((1, 2 * D), jnp.float32),
                   jax.ShapeDtypeStruct((N, 2 * D), jnp.float32)],
    )(xa, xm)


def _vgather(v, idx):
    return lax.gather(
        v, idx[:, None],
        lax.GatherDimensionNumbers(offset_dims=(), collapsed_slice_dims=(0,),
                                   start_index_map=(0,)),
        (1,), mode=lax.GatherScatterMode.PROMISE_IN_BOUNDS)


CMAX = E // CE - 1


def _sc_edge_body(qp_hbm, kv_hbm, ef_hbm, src_hbm, rowlo_hbm, rowhi_hbm, bounds_hbm,
             idx_hbm, out_hbm, rowlo_v, rowhi_v, bounds_v, idx_v, src_v, kv_v,
             ef_v, qp_v, out_v, sem0, sem1):
    wid = lax.axis_index("s") * 2 + lax.axis_index("c")
    pltpu.sync_copy(rowlo_hbm, rowlo_v)
    pltpu.sync_copy(rowhi_hbm, rowhi_v)
    pltpu.sync_copy(bounds_hbm, bounds_v)
    pltpu.sync_copy(idx_hbm, idx_v)
    bv = bounds_v[pl.ds(wid, 16)]
    b0 = bv[0]
    b1 = bv[1]

    idx_hi = idx_v[0, :]
    idx_lo2 = idx_v[1, :]
    idxp = [idx_v[2 + ii, :] for ii in range(8)]

    def _drain(pref, buf, sem):
        pltpu.make_async_copy(kv_hbm.at[src_v.at[buf]], kv_v.at[buf], sem).wait()
        pltpu.make_async_copy(ef_hbm.at[pl.ds(pref * CE, CE)], ef_v.at[buf],
                              sem).wait()

    def _cold_load(c, buf):
        pltpu.sync_copy(src_hbm.at[pl.ds(c * CE, CE)], src_v.at[buf])
        pltpu.sync_copy(kv_hbm.at[src_v.at[buf]], kv_v.at[buf])
        pltpu.sync_copy(ef_hbm.at[pl.ds(c * CE, CE)], ef_v.at[buf])

    def _prefetch(c1, buf, sem):
        pltpu.sync_copy(src_hbm.at[pl.ds(c1 * CE, CE)], src_v.at[buf])
        pltpu.async_copy(kv_hbm.at[src_v.at[buf]], kv_v.at[buf], sem)
        pltpu.async_copy(ef_hbm.at[pl.ds(c1 * CE, CE)], ef_v.at[buf], sem)

    def b_body(b, bc):
        pltpu.sync_copy(qp_hbm.at[pl.ds(b * NB, NB)], qp_v)

        def r_body(r, bc):
            n = b * NB + r
            jlo = rowlo_v[pl.ds(n, 16)][0]
            jhi = rowhi_v[pl.ds(n, 16)][0]
            qregs = [qp_v[r, pl.ds(ii * 16, 16)] for ii in range(16)]

            zero = jnp.zeros((16,), jnp.float32)
            minit = jnp.full((16,), -1e30, jnp.float32)
            init = bc + (minit, zero) + tuple([zero] * 16)

            def e_body(j, carry):
                loaded_c, pref, m, den = carry[0], carry[1], carry[2], carry[3]
                accs = carry[4:]
                c = j // CE

                @pl.when(c != loaded_c)
                def _switch():
                    @pl.when(pref >= 0)
                    def _():
                        @pl.when((pref & 1) == 0)
                        def _():
                            _drain(pref, 0, sem0)

                        @pl.when((pref & 1) == 1)
                        def _():
                            _drain(pref, 1, sem1)

                    @pl.when(pref != c)
                    def _():
                        @pl.when((c & 1) == 0)
                        def _():
                            _cold_load(c, 0)

                        @pl.when((c & 1) == 1)
                        def _():
                            _cold_load(c, 1)

                    @pl.when(c + 1 <= CMAX)
                    def _():
                        @pl.when((c & 1) == 0)
                        def _():
                            _prefetch(c + 1, 1, sem1)

                        @pl.when((c & 1) == 1)
                        def _():
                            _prefetch(c + 1, 0, sem0)

                new_pref = jnp.where(c != loaded_c,
                                     jnp.where(c + 1 <= CMAX, c + 1,
                                               jnp.int32(-2)),
                                     pref)

                bb = c & 1
                i = j - c * CE
                efv = ef_v[bb, i, :]
                eps = [_vgather(efv, idxp[ii]) for ii in range(8)]
                sacc = zero
                for ii in range(8):
                    sacc = sacc + kv_v[bb, i, pl.ds(ii * 16, 16)] * qregs[ii]
                for ii in range(8):
                    sacc = sacc + eps[ii] * qregs[8 + ii]
                s = sacc + _vgather(sacc, idx_hi)

                mn = jnp.maximum(m, s)
                scale = jnp.exp(m - mn)
                p = jnp.exp(s - mn)
                den2 = den * scale + p
                pp = _vgather(p, idx_lo2)
                scp = _vgather(scale, idx_lo2)
                naccs = []
                for ii in range(8):
                    naccs.append(accs[ii] * scp
                                 + kv_v[bb, i, pl.ds(D + ii * 16, 16)] * pp)
                for ii in range(8):
                    naccs.append(accs[8 + ii] * scp + eps[ii] * pp)
                return (c, new_pref, mn, den2) + tuple(naccs)

            fin = lax.fori_loop(jlo, jhi, e_body, init)
            den = fin[3]
            accs = fin[4:]
            rden = 1.0 / (den + 1e-9)
            rpp = _vgather(rden, idx_lo2)
            for ii in range(16):
                out_v[r, pl.ds(ii * 16, 16)] = accs[ii] * rpp
            return (fin[0], fin[1])

        bc = lax.fori_loop(0, NB, r_body, bc)
        pltpu.sync_copy(out_v, out_hbm.at[pl.ds(b * NB, NB)])
        return bc

    bcf = lax.fori_loop(b0, b1, b_body, (jnp.int32(-1), jnp.int32(-2)))
    pref_end = bcf[1]

    @pl.when(pref_end >= 0)
    def _final_drain():
        @pl.when((pref_end & 1) == 0)
        def _():
            _drain(pref_end, 0, sem0)

        @pl.when((pref_end & 1) == 1)
        def _():
            _drain(pref_end, 1, sem1)


@functools.lru_cache(maxsize=None)
def _sc_edge():
    mesh = plsc.VectorSubcoreMesh(core_axis_name="c", subcore_axis_name="s")
    return pl.kernel(
        _sc_edge_body,
        mesh=mesh,
        out_type=jax.ShapeDtypeStruct((N, 2 * D), jnp.float32),
        scratch_types=[
            pltpu.VMEM((ROWP,), jnp.int32),
            pltpu.VMEM((ROWP,), jnp.int32),
            pltpu.VMEM((48,), jnp.int32),
            pltpu.VMEM((16, 16), jnp.int32),
            pltpu.VMEM((2, CE), jnp.int32),
            pltpu.VMEM((2, CE, 2 * D), jnp.float32),
            pltpu.VMEM((2, CE, ED), jnp.float32),
            pltpu.VMEM((NB, 2 * D), jnp.float32),
            pltpu.VMEM((NB, 2 * D), jnp.float32),
            pltpu.SemaphoreType.DMA,
            pltpu.SemaphoreType.DMA,
        ],
    )


def _bounds(cum_end, total):
    targets = (jnp.arange(33, dtype=jnp.int32) * total) // 32
    bnd = jnp.searchsorted(cum_end, targets, side="left").astype(jnp.int32) // NB
    bnd = bnd.at[0].set(0).at[32].set(NBL)
    bnd = jnp.concatenate([bnd, jnp.zeros((15,), jnp.int32)])
    return bnd


def kernel(input, unit_type, edge_index, edge_feature, emb_mono, emb_atom,
           Wq_a, Wk_a, Wv_a, Wo_a, We_a, Wq_m, Wk_m, Wv_m, Wo_m, We_m):
    src = edge_index[0].astype(jnp.int32)
    dst = edge_index[1].astype(jnp.int32)
    ut = unit_type.astype(jnp.int32)

    mono_e = (ut[src] < U) & (ut[dst] < U)
    key2 = dst * 2 + (1 - mono_e.astype(jnp.int32))
    perm = jnp.argsort(key2)
    srcP = src[perm].astype(jnp.int32)
    efP = edge_feature[perm]
    cntA = jnp.bincount(dst, length=N).astype(jnp.int32)
    cntM = jnp.bincount(jnp.where(mono_e, dst, N), length=N + 1)[:N].astype(jnp.int32)
    cumA = jnp.cumsum(cntA).astype(jnp.int32)
    rowlo = cumA - cntA
    rowhiM = rowlo + cntM
    cumM = jnp.cumsum(cntM).astype(jnp.int32)
    bndA = _bounds(cumA, jnp.int32(E))
    bndM = _bounds(cumM, cumM[N - 1])
    zpad = jnp.zeros((16,), jnp.int32)
    rowlo_p = jnp.concatenate([rowlo, zpad])
    cumA_p = jnp.concatenate([cumA, zpad])
    rowhiM_p = jnp.concatenate([rowhiM, zpad])
    idxtab = jnp.asarray(IDX_TAB)

    WQPs, WKVs, WBs = [], [], []
    for i in range(L):
        for (Wq, Wk, Wv, We, Wo) in ((Wq_a[i], Wk_a[i], Wv_a[i], We_a[i], Wo_a[i]),
                                     (Wq_m[i], Wk_m[i], Wv_m[i], We_m[i], Wo_m[i])):
            WQP, WKV, WB = _fold_weights(Wq, Wk, Wv, We, Wo)
            WQPs.append(WQP)
            WKVs.append(WKV)
            WBs.append(WB)

    ut2 = ut.reshape(N, 1)
    TQP_a = emb_atom @ WQPs[0]
    TKV_a = emb_atom @ WKVs[0]
    TQP_m = emb_mono @ WQPs[1]
    TKV_m = emb_mono @ WKVs[1]
    xa, qpa, kva = _tc_emb(ut2, emb_atom, TQP_a, TKV_a)
    xm, qpm, kvm = _tc_emb(ut2, emb_mono, TQP_m, TKV_m)

    for i in range(L):
        if i > 0:
            qpa, kva = _tc_proj(xa, WQPs[2 * i], WKVs[2 * i])
            qpm, kvm = _tc_proj(xm, WQPs[2 * i + 1], WKVs[2 * i + 1])
        agga = _sc_edge()(qpa, kva, efP, srcP, rowlo_p, cumA_p, bndA, idxtab)
        aggm = _sc_edge()(qpm, kvm, efP, srcP, rowlo_p, rowhiM_p, bndM, idxtab)
        xa = _tc_out(agga, xa, WBs[2 * i])
        xm = _tc_out(aggm, xm, WBs[2 * i + 1])

    gf, nf = _readout(xa, xm)
    return (gf, nf)


# speedup vs baseline: 1.2047x; 1.0039x over previous
import functools

import jax
import jax.numpy as jnp
import numpy as np
from jax import lax
from jax.experimental import pallas as pl
from jax.experimental.pallas import tpu as pltpu
from jax.experimental.pallas import tpu_sc as plsc

N = 10000
E = 320000
D = 128
ED = 16
L = 4
H = 8
V = 256
U = 150
DH = D // H
CE = 128
NB = 16
NT = 32
NBL = N // NB
ROWP = N + 16

_pos = np.arange(D)
PERM_T = ((_pos % H) * DH + _pos // H).astype(np.int32)

_lane = np.arange(16)
IDX_TAB = np.zeros((16, 16), np.int32)
IDX_TAB[0] = (_lane + 8) % 16
IDX_TAB[1] = _lane % 8
for _ii in range(8):
    IDX_TAB[2 + _ii] = 2 * _ii + _lane // 8


def _we2(We):
    c = np.arange(D)
    d = np.arange(D)
    mask = (d[:, None] // DH) == (c[None, :] // DH)
    A = We[c % ED, :]
    return A.T * mask


def _we3(We):
    r = np.arange(D)
    c = np.arange(D)
    mask = (r[:, None] // DH) == (c[None, :] // DH)
    A2 = We[r % ED, :]
    return A2 * mask


def _fold_weights(Wq, Wk, Wv, We, Wo):
    We2 = _we2(We)
    WqT = (0.25 * Wq)[:, PERM_T]
    WPT = (0.25 * (Wq @ We2))[:, PERM_T]
    WQP = jnp.concatenate([WqT, WPT], axis=1)
    WKV = jnp.concatenate([Wk[:, PERM_T], Wv[:, PERM_T]], axis=1)
    WB = jnp.concatenate([Wo[PERM_T, :], (_we3(We) @ Wo)[PERM_T, :]], axis=0)
    return WQP, WKV, WB


def _tc_proj_body(x_ref, wqp_ref, wkv_ref, qp_ref, kv_ref):
    x = x_ref[...]
    qp_ref[...] = jnp.dot(x, wqp_ref[...], preferred_element_type=jnp.float32)
    kv_ref[...] = jnp.dot(x, wkv_ref[...], preferred_element_type=jnp.float32)


def _tc_proj(x, WQP, WKV):
    BN_ = 1000
    return pl.pallas_call(
        _tc_proj_body,
        grid=(N // BN_,),
        in_specs=[pl.BlockSpec((BN_, D), lambda i: (i, 0)),
                  pl.BlockSpec((D, 2 * D), lambda i: (0, 0)),
                  pl.BlockSpec((D, 2 * D), lambda i: (0, 0))],
        out_specs=[pl.BlockSpec((BN_, 2 * D), lambda i: (i, 0)),
                   pl.BlockSpec((BN_, 2 * D), lambda i: (i, 0))],
        out_shape=[jax.ShapeDtypeStruct((N, 2 * D), jnp.float32),
                   jax.ShapeDtypeStruct((N, 2 * D), jnp.float32)],
    )(x, WQP, WKV)


def _tc_emb_body(ut_ref, tab_ref, tqp_ref, tkv_ref, x0_ref, qp_ref, kv_ref):
    oh = (ut_ref[...] == lax.broadcasted_iota(jnp.int32, (ut_ref.shape[0], V), 1)
          ).astype(jnp.float32)
    x0_ref[...] = jnp.dot(oh, tab_ref[...], preferred_element_type=jnp.float32)
    qp_ref[...] = jnp.dot(oh, tqp_ref[...], preferred_element_type=jnp.float32)
    kv_ref[...] = jnp.dot(oh, tkv_ref[...], preferred_element_type=jnp.float32)


def _tc_emb(ut2, TAB, TQP, TKV):
    BN_ = 1000
    return pl.pallas_call(
        _tc_emb_body,
        grid=(N // BN_,),
        in_specs=[pl.BlockSpec((BN_, 1), lambda i: (i, 0)),
                  pl.BlockSpec((V, D), lambda i: (0, 0)),
                  pl.BlockSpec((V, 2 * D), lambda i: (0, 0)),
                  pl.BlockSpec((V, 2 * D), lambda i: (0, 0))],
        out_specs=[pl.BlockSpec((BN_, D), lambda i: (i, 0)),
                   pl.BlockSpec((BN_, 2 * D), lambda i: (i, 0)),
                   pl.BlockSpec((BN_, 2 * D), lambda i: (i, 0))],
        out_shape=[jax.ShapeDtypeStruct((N, D), jnp.float32),
                   jax.ShapeDtypeStruct((N, 2 * D), jnp.float32),
                   jax.ShapeDtypeStruct((N, 2 * D), jnp.float32)],
    )(ut2, TAB, TQP, TKV)


def _tc_out_body(aggb_ref, x_ref, wb_ref, xn_ref):
    y = jnp.dot(aggb_ref[...], wb_ref[...], preferred_element_type=jnp.float32)
    xn_ref[...] = jnp.maximum(y, 0.0) + x_ref[...]


def _tc_out(aggb, x, WB):
    BN_ = 1000
    return pl.pallas_call(
        _tc_out_body,
        grid=(N // BN_,),
        in_specs=[pl.BlockSpec((BN_, 2 * D), lambda i: (i, 0)),
                  pl.BlockSpec((BN_, D), lambda i: (i, 0)),
                  pl.BlockSpec((2 * D, D), lambda i: (0, 0))],
        out_specs=pl.BlockSpec((BN_, D), lambda i: (i, 0)),
        out_shape=jax.ShapeDtypeStruct((N, D), jnp.float32),
    )(aggb, x, WB)


def _readout_body(xa_ref, xm_ref, gf_ref, nf_ref):
    i = pl.program_id(0)

    @pl.when(i == 0)
    def _init():
        gf_ref[...] = jnp.zeros_like(gf_ref)

    a = xa_ref[...]
    m = xm_ref[...]
    nf_ref[:, 0:D] = a
    nf_ref[:, D:2 * D] = m
    gf_ref[:, 0:D] += a.sum(axis=0, keepdims=True)
    gf_ref[:, D:2 * D] += m.sum(axis=0, keepdims=True)


def _readout(xa, xm):
    BN_ = 1000
    return pl.pallas_call(
        _readout_body,
        grid=(N // BN_,),
        in_specs=[pl.BlockSpec((BN_, D), lambda i: (i, 0)),
                  pl.BlockSpec((BN_, D), lambda i: (i, 0))],
        out_specs=[pl.BlockSpec((1, 2 * D), lambda i: (0, 0)),
                   pl.BlockSpec((BN_, 2 * D), lambda i: (i, 0))],
        out_shape=[jax.ShapeDtypeStruct((1, 2 * D), jnp.float32),
                   jax.ShapeDtypeStruct((N, 2 * D), jnp.float32)],
    )(xa, xm)


def _vgather(v, idx):
    return lax.gather(
        v, idx[:, None],
        lax.GatherDimensionNumbers(offset_dims=(), collapsed_slice_dims=(0,),
                                   start_index_map=(0,)),
        (1,), mode=lax.GatherScatterMode.PROMISE_IN_BOUNDS)


CMAX = E // CE - 1


def _sc_edge_body(qp_hbm, kv_hbm, ef_hbm, src_hbm, rowlo_hbm, rowhi_hbm, bounds_hbm,
             idx_hbm, out_hbm, rowlo_v, rowhi_v, bounds_v, idx_v, src_v, kv_v,
             ef_v, qp_v, out_v, sem0, sem1):
    wid = lax.axis_index("s") * 2 + lax.axis_index("c")
    pltpu.sync_copy(rowlo_hbm, rowlo_v)
    pltpu.sync_copy(rowhi_hbm, rowhi_v)
    pltpu.sync_copy(bounds_hbm, bounds_v)
    pltpu.sync_copy(idx_hbm, idx_v)
    bv = bounds_v[pl.ds(wid, 16)]
    b0 = bv[0]
    b1 = bv[1]

    idx_hi = idx_v[0, :]
    idx_lo2 = idx_v[1, :]
    idxp = [idx_v[2 + ii, :] for ii in range(8)]

    def _drain(pref, buf, sem):
        pltpu.make_async_copy(kv_hbm.at[src_v.at[buf]], kv_v.at[buf], sem).wait()
        pltpu.make_async_copy(ef_hbm.at[pl.ds(pref * CE, CE)], ef_v.at[buf],
                              sem).wait()

    def _cold_load(c, buf):
        pltpu.sync_copy(src_hbm.at[pl.ds(c * CE, CE)], src_v.at[buf])
        pltpu.sync_copy(kv_hbm.at[src_v.at[buf]], kv_v.at[buf])
        pltpu.sync_copy(ef_hbm.at[pl.ds(c * CE, CE)], ef_v.at[buf])

    def _prefetch(c1, buf, sem):
        pltpu.sync_copy(src_hbm.at[pl.ds(c1 * CE, CE)], src_v.at[buf])
        pltpu.async_copy(kv_hbm.at[src_v.at[buf]], kv_v.at[buf], sem)
        pltpu.async_copy(ef_hbm.at[pl.ds(c1 * CE, CE)], ef_v.at[buf], sem)

    def b_body(b, bc):
        pltpu.sync_copy(qp_hbm.at[pl.ds(b * NB, NB)], qp_v)

        def r_body(r, bc):
            n = b * NB + r
            jlo = rowlo_v[pl.ds(n, 16)][0]
            jhi = rowhi_v[pl.ds(n, 16)][0]
            qregs = [qp_v[r, pl.ds(ii * 16, 16)] for ii in range(16)]

            zero = jnp.zeros((16,), jnp.float32)
            minit = jnp.full((16,), -1e30, jnp.float32)
            init = bc + (minit, zero) + tuple([zero] * 16)

            def e_body(j, carry):
                loaded_c, pref, m, den = carry[0], carry[1], carry[2], carry[3]
                accs = carry[4:]
                c = j // CE

                @pl.when(c != loaded_c)
                def _switch():
                    @pl.when(pref >= 0)
                    def _():
                        @pl.when((pref & 1) == 0)
                        def _():
                            _drain(pref, 0, sem0)

                        @pl.when((pref & 1) == 1)
                        def _():
                            _drain(pref, 1, sem1)

                    @pl.when(pref != c)
                    def _():
                        @pl.when((c & 1) == 0)
                        def _():
                            _cold_load(c, 0)

                        @pl.when((c & 1) == 1)
                        def _():
                            _cold_load(c, 1)

                    @pl.when(c + 1 <= CMAX)
                    def _():
                        @pl.when((c & 1) == 0)
                        def _():
                            _prefetch(c + 1, 1, sem1)

                        @pl.when((c & 1) == 1)
                        def _():
                            _prefetch(c + 1, 0, sem0)

                new_pref = jnp.where(c != loaded_c,
                                     jnp.where(c + 1 <= CMAX, c + 1,
                                               jnp.int32(-2)),
                                     pref)

                bb = c & 1
                i = j - c * CE
                efv = ef_v[bb, i, :]
                eps = [_vgather(efv, idxp[ii]) for ii in range(8)]
                sacc = zero
                for ii in range(8):
                    sacc = sacc + kv_v[bb, i, pl.ds(ii * 16, 16)] * qregs[ii]
                for ii in range(8):
                    sacc = sacc + eps[ii] * qregs[8 + ii]
                s = sacc + _vgather(sacc, idx_hi)

                mn = jnp.maximum(m, s)
                scale = jnp.exp(m - mn)
                p = jnp.exp(s - mn)
                den2 = den * scale + p
                naccs = []
                for ii in range(8):
                    naccs.append(accs[ii] * scale
                                 + kv_v[bb, i, pl.ds(D + ii * 16, 16)] * p)
                for ii in range(8):
                    naccs.append(accs[8 + ii] * scale + eps[ii] * p)
                return (c, new_pref, mn, den2) + tuple(naccs)

            fin = lax.fori_loop(jlo, jhi, e_body, init)
            den = fin[3]
            accs = fin[4:]
            rden = 1.0 / (den + 1e-9)
            for ii in range(16):
                out_v[r, pl.ds(ii * 16, 16)] = accs[ii] * rden
            return (fin[0], fin[1])

        bc = lax.fori_loop(0, NB, r_body, bc)
        pltpu.sync_copy(out_v, out_hbm.at[pl.ds(b * NB, NB)])
        return bc

    bcf = lax.fori_loop(b0, b1, b_body, (jnp.int32(-1), jnp.int32(-2)))
    pref_end = bcf[1]

    @pl.when(pref_end >= 0)
    def _final_drain():
        @pl.when((pref_end & 1) == 0)
        def _():
            _drain(pref_end, 0, sem0)

        @pl.when((pref_end & 1) == 1)
        def _():
            _drain(pref_end, 1, sem1)


@functools.lru_cache(maxsize=None)
def _sc_edge():
    mesh = plsc.VectorSubcoreMesh(core_axis_name="c", subcore_axis_name="s")
    return pl.kernel(
        _sc_edge_body,
        mesh=mesh,
        out_type=jax.ShapeDtypeStruct((N, 2 * D), jnp.float32),
        scratch_types=[
            pltpu.VMEM((ROWP,), jnp.int32),
            pltpu.VMEM((ROWP,), jnp.int32),
            pltpu.VMEM((48,), jnp.int32),
            pltpu.VMEM((16, 16), jnp.int32),
            pltpu.VMEM((2, CE), jnp.int32),
            pltpu.VMEM((2, CE, 2 * D), jnp.float32),
            pltpu.VMEM((2, CE, ED), jnp.float32),
            pltpu.VMEM((NB, 2 * D), jnp.float32),
            pltpu.VMEM((NB, 2 * D), jnp.float32),
            pltpu.SemaphoreType.DMA,
            pltpu.SemaphoreType.DMA,
        ],
    )


def _bounds(cum_end, total):
    targets = (jnp.arange(33, dtype=jnp.int32) * total) // 32
    bnd = jnp.searchsorted(cum_end, targets, side="left").astype(jnp.int32) // NB
    bnd = bnd.at[0].set(0).at[32].set(NBL)
    bnd = jnp.concatenate([bnd, jnp.zeros((15,), jnp.int32)])
    return bnd


def kernel(input, unit_type, edge_index, edge_feature, emb_mono, emb_atom,
           Wq_a, Wk_a, Wv_a, Wo_a, We_a, Wq_m, Wk_m, Wv_m, Wo_m, We_m):
    src = edge_index[0].astype(jnp.int32)
    dst = edge_index[1].astype(jnp.int32)
    ut = unit_type.astype(jnp.int32)

    mono_e = (ut[src] < U) & (ut[dst] < U)
    key2 = dst * 2 + (1 - mono_e.astype(jnp.int32))
    perm = jnp.argsort(key2)
    srcP = src[perm].astype(jnp.int32)
    efP = edge_feature[perm]
    cntA = jnp.bincount(dst, length=N).astype(jnp.int32)
    cntM = jnp.bincount(jnp.where(mono_e, dst, N), length=N + 1)[:N].astype(jnp.int32)
    cumA = jnp.cumsum(cntA).astype(jnp.int32)
    rowlo = cumA - cntA
    rowhiM = rowlo + cntM
    cumM = jnp.cumsum(cntM).astype(jnp.int32)
    bndA = _bounds(cumA, jnp.int32(E))
    bndM = _bounds(cumM, cumM[N - 1])
    zpad = jnp.zeros((16,), jnp.int32)
    rowlo_p = jnp.concatenate([rowlo, zpad])
    cumA_p = jnp.concatenate([cumA, zpad])
    rowhiM_p = jnp.concatenate([rowhiM, zpad])
    idxtab = jnp.asarray(IDX_TAB)

    WQPs, WKVs, WBs = [], [], []
    for i in range(L):
        for (Wq, Wk, Wv, We, Wo) in ((Wq_a[i], Wk_a[i], Wv_a[i], We_a[i], Wo_a[i]),
                                     (Wq_m[i], Wk_m[i], Wv_m[i], We_m[i], Wo_m[i])):
            WQP, WKV, WB = _fold_weights(Wq, Wk, Wv, We, Wo)
            WQPs.append(WQP)
            WKVs.append(WKV)
            WBs.append(WB)

    ut2 = ut.reshape(N, 1)
    TQP_a = emb_atom @ WQPs[0]
    TKV_a = emb_atom @ WKVs[0]
    TQP_m = emb_mono @ WQPs[1]
    TKV_m = emb_mono @ WKVs[1]
    xa, qpa, kva = _tc_emb(ut2, emb_atom, TQP_a, TKV_a)
    xm, qpm, kvm = _tc_emb(ut2, emb_mono, TQP_m, TKV_m)

    for i in range(L):
        if i > 0:
            qpa, kva = _tc_proj(xa, WQPs[2 * i], WKVs[2 * i])
            qpm, kvm = _tc_proj(xm, WQPs[2 * i + 1], WKVs[2 * i + 1])
        agga = _sc_edge()(qpa, kva, efP, srcP, rowlo_p, cumA_p, bndA, idxtab)
        aggm = _sc_edge()(qpm, kvm, efP, srcP, rowlo_p, rowhiM_p, bndM, idxtab)
        xa = _tc_out(agga, xa, WBs[2 * i])
        xm = _tc_out(aggm, xm, WBs[2 * i + 1])

    gf, nf = _readout(xa, xm)
    return (gf, nf)
